# Initial kernel scaffold; baseline (speedup 1.0000x reference)
import jax
import jax.numpy as jnp
from jax.experimental import pallas as pl


def kernel(z, emb):
    raise NotImplementedError("write your pallas kernel here")


# speedup vs baseline: 1.5337x; 1.5337x over previous
import jax
import jax.numpy as jnp
from jax.experimental import pallas as pl
from jax.experimental.pallas import tpu as pltpu

N_CODES = 1024
EDIM = 64
BETA = 0.25


def _vq_kernel(zp_ref, emb_ref, zpsq_ref, embsq_ref,
               zq_ref, loss_ref, perp_ref, counts_ref, acc_ref):
    i = pl.program_id(0)
    nsteps = pl.num_programs(0)
    zp = zp_ref[...]
    emb = emb_ref[...]
    TB = zp.shape[0]
    K = emb.shape[0]

    dot = jax.lax.dot_general(zp, emb, (((1,), (1,)), ((), ())))
    d = (zpsq_ref[...] + embsq_ref[...]) - 2.0 * dot

    dmin = jnp.min(d, axis=1, keepdims=True)
    iota_k = jax.lax.broadcasted_iota(jnp.int32, (TB, K), 1)
    hit = d == dmin
    idx = jnp.min(jnp.where(hit, iota_k, K), axis=1)

    p = (iota_k == idx[:, None]).astype(jnp.float32)
    zq = jax.lax.dot_general(p, emb, (((1,), (0,)), ((), ())))

    zq_ref[...] = zp + (zq - zp)

    diff = zq - zp
    part = jnp.sum(diff * diff)
    cnt = jnp.sum(p, axis=0, keepdims=True)

    @pl.when(i == 0)
    def _():
        acc_ref[0, 0] = 0.0
        counts_ref[...] = jnp.zeros_like(counts_ref)

    acc_ref[0, 0] += part
    counts_ref[...] += cnt

    @pl.when(i == nsteps - 1)
    def _():
        total_rows = nsteps * TB
        m = acc_ref[0, 0] / (total_rows * EDIM)
        loss_ref[...] = jnp.reshape(m + BETA * m, (1, 1))
        e_mean = counts_ref[...] / total_rows
        plogp = e_mean * jnp.log(e_mean + 1e-10)
        perp_ref[...] = jnp.reshape(jnp.exp(-jnp.sum(plogp)), (1, 1))


def kernel(z, emb):
    N, W, T = z.shape
    K = emb.shape[0]
    zp = jnp.transpose(z, (0, 2, 1)).reshape(-1, W)
    zpsq = jnp.sum(zp ** 2, axis=1, keepdims=True)
    embsq = jnp.sum(emb ** 2, axis=1)[None, :]
    rows = N * T
    TB = 2048
    grid = rows // TB
    zq, loss, perp = pl.pallas_call(
        _vq_kernel,
        grid=(grid,),
        in_specs=[
            pl.BlockSpec((TB, W), lambda i: (i, 0)),
            pl.BlockSpec((K, W), lambda i: (0, 0)),
            pl.BlockSpec((TB, 1), lambda i: (i, 0)),
            pl.BlockSpec((1, K), lambda i: (0, 0)),
        ],
        out_specs=[
            pl.BlockSpec((TB, W), lambda i: (i, 0)),
            pl.BlockSpec((1, 1), lambda i: (0, 0)),
            pl.BlockSpec((1, 1), lambda i: (0, 0)),
        ],
        out_shape=[
            jax.ShapeDtypeStruct((rows, W), jnp.float32),
            jax.ShapeDtypeStruct((1, 1), jnp.float32),
            jax.ShapeDtypeStruct((1, 1), jnp.float32),
        ],
        scratch_shapes=[
            pltpu.VMEM((1, K), jnp.float32),
            pltpu.SMEM((1, 1), jnp.float32),
        ],
        compiler_params=pltpu.CompilerParams(
            dimension_semantics=("arbitrary",)),
    )(zp, emb, zpsq, embsq)
    zq_out = jnp.transpose(zq.reshape(N, T, W), (0, 2, 1))
    return zq_out, loss[0, 0], perp[0, 0]


# speedup vs baseline: 2.0181x; 1.3158x over previous
import jax
import jax.numpy as jnp
from jax.experimental import pallas as pl
from jax.experimental.pallas import tpu as pltpu

N_CODES = 1024
EDIM = 64
BETA = 0.25


def _vq_kernel(z_ref, emb_ref, zpsq_ref, embsq_ref,
               zq_ref, loss_ref, perp_ref, counts_ref, acc_ref):
    i = pl.program_id(0)
    nsteps = pl.num_programs(0)
    z_n = z_ref[0]
    emb = emb_ref[...]
    T = z_n.shape[1]
    K = emb.shape[0]

    zp = z_n.T

    dot = jax.lax.dot_general(zp, emb, (((1,), (1,)), ((), ())))
    d = (zpsq_ref[...] + embsq_ref[...]) - 2.0 * dot

    dmin = jnp.min(d, axis=1, keepdims=True)
    iota_k = jax.lax.broadcasted_iota(jnp.int32, (T, K), 1)
    idx = jnp.min(jnp.where(d == dmin, iota_k, K), axis=1)

    p = (iota_k == idx[:, None]).astype(jnp.float32)
    zqt = jax.lax.dot_general(emb, p, (((0,), (1,)), ((), ())))

    zq_ref[0] = z_n + (zqt - z_n)

    diff = zqt - z_n
    part = jnp.sum(diff * diff)
    cnt = jnp.sum(p, axis=0, keepdims=True)

    @pl.when(i == 0)
    def _():
        acc_ref[0, 0] = 0.0
        counts_ref[...] = jnp.zeros_like(counts_ref)

    acc_ref[0, 0] += part
    counts_ref[...] += cnt

    @pl.when(i == nsteps - 1)
    def _():
        total_rows = nsteps * T
        m = acc_ref[0, 0] / (total_rows * EDIM)
        loss_ref[...] = jnp.reshape(m + BETA * m, (1, 1))
        e_mean = counts_ref[...] / total_rows
        plogp = e_mean * jnp.log(e_mean + 1e-10)
        perp_ref[...] = jnp.reshape(jnp.exp(-jnp.sum(plogp)), (1, 1))


def kernel(z, emb):
    N, W, T = z.shape
    K = emb.shape[0]
    zpsq = jnp.sum(z ** 2, axis=1).reshape(-1, 1)
    embsq = jnp.sum(emb ** 2, axis=1)[None, :]
    zq, loss, perp = pl.pallas_call(
        _vq_kernel,
        grid=(N,),
        in_specs=[
            pl.BlockSpec((1, W, T), lambda i: (i, 0, 0)),
            pl.BlockSpec((K, W), lambda i: (0, 0)),
            pl.BlockSpec((T, 1), lambda i: (i, 0)),
            pl.BlockSpec((1, K), lambda i: (0, 0)),
        ],
        out_specs=[
            pl.BlockSpec((1, W, T), lambda i: (i, 0, 0)),
            pl.BlockSpec((1, 1), lambda i: (0, 0)),
            pl.BlockSpec((1, 1), lambda i: (0, 0)),
        ],
        out_shape=[
            jax.ShapeDtypeStruct((N, W, T), jnp.float32),
            jax.ShapeDtypeStruct((1, 1), jnp.float32),
            jax.ShapeDtypeStruct((1, 1), jnp.float32),
        ],
        scratch_shapes=[
            pltpu.VMEM((1, K), jnp.float32),
            pltpu.SMEM((1, 1), jnp.float32),
        ],
        compiler_params=pltpu.CompilerParams(
            dimension_semantics=("arbitrary",)),
    )(z, emb, zpsq, embsq)
    return zq, loss[0, 0], perp[0, 0]
